# baseline (speedup 1.0000x reference)
import functools

import jax
import jax.numpy as jnp
from jax import lax
from jax.experimental import pallas as pl
from jax.experimental.pallas import tpu as pltpu
from jax.experimental.pallas import tpu_sc as plsc

_VOCAB = 1000
_VP = 1024
_D = 512
_B = 128
_S = 2048
_K = 204
_KPAD = 208
_NC = 1
_NW = 16 * _NC
_ROWS_PER = _B // _NW
_L = 16


def _vocab_body(sc_ref, tab_ref, a1_ref, ba1_ref, a2_ref, ba2_ref,
                rank_ref, g_ref):
    sc = sc_ref[...]
    bits = lax.bitcast_convert_type(sc, jnp.int32)
    bits_r = jnp.transpose(bits, (1, 0))
    gt = bits > bits_r
    real_row = lax.broadcasted_iota(jnp.int32, (_VP, _VP), 1) < _VOCAB
    nsmaller = jnp.sum(jnp.where(gt & real_row, 1, 0).astype(jnp.int32),
                       axis=1, keepdims=True)
    rank_ref[...] = (_VOCAB - 1) - nsmaller

    tab = tab_ref[...]
    a1 = jax.nn.relu(jnp.dot(tab, a1_ref[...], precision=lax.Precision.HIGHEST,
                             preferred_element_type=jnp.float32) + ba1_ref[...])
    g_ref[...] = jnp.dot(a1, a2_ref[...], precision=lax.Precision.HIGHEST,
                         preferred_element_type=jnp.float32) + ba2_ref[...]


_vocab_kernel = pl.pallas_call(
    _vocab_body,
    out_shape=(
        jax.ShapeDtypeStruct((_VP, 1), jnp.int32),
        jax.ShapeDtypeStruct((_VP, _D), jnp.float32),
    ),
)


def _sc_body(x_hbm, vrank_hbm, vscore_hbm,
             scores_hbm, topidx_hbm, cnt_hbm,
             *scratch):
    R = _ROWS_PER
    vrank, vscore = scratch[0], scratch[1]
    x_rows = scratch[2:2 + R]
    rank_rows = scratch[2 + R:2 + 2 * R]
    score_rows = scratch[2 + 2 * R:2 + 3 * R]
    hists = scratch[2 + 3 * R:2 + 4 * R]
    cnt_toks = scratch[2 + 4 * R:2 + 5 * R]
    out_idxs = scratch[2 + 5 * R:2 + 6 * R]
    sel_toks = scratch[2 + 6 * R:2 + 7 * R]
    sem = scratch[2 + 7 * R]

    cid = lax.axis_index("c")
    sid = lax.axis_index("s")
    wid = sid * _NC + cid
    b0 = wid * R
    in_dmas = [pltpu.async_copy(x_hbm.at[b0 + i], x_rows[i], sem)
               for i in range(R)]
    pltpu.sync_copy(vrank_hbm, vrank)
    pltpu.sync_copy(vscore_hbm, vscore)
    for d in in_dmas:
        d.wait()

    @plsc.parallel_loop(0, _VP // _L, 1, unroll=4)
    def zero_body(j):
        for i in range(R):
            hists[i][pl.ds(j * _L, _L)] = jnp.zeros((_L,), jnp.int32)
            cnt_toks[i][pl.ds(j * _L, _L)] = jnp.zeros((_L,), jnp.float32)

    @plsc.parallel_loop(0, _S // _L, 1, unroll=2)
    def p1(c):
        off = c * _L
        toks = [x_rows[i][pl.ds(off, _L)] for i in range(R)]
        rs = [plsc.load_gather(vrank, [toks[i]]) for i in range(R)]
        scs = [plsc.load_gather(vscore, [toks[i]]) for i in range(R)]
        scans = [plsc.scan_count(rs[i]) for i in range(R)]
        for i in range(R):
            rank_rows[i][pl.ds(off, _L)] = rs[i]
        for i in range(R):
            score_rows[i][pl.ds(off, _L)] = scs[i]
        for i in range(R):
            cnt, last = scans[i]
            plsc.addupdate_scatter(hists[i], [rs[i]], cnt, mask=last)
    score_dmas = [pltpu.async_copy(score_rows[i], scores_hbm.at[b0 + i], sem)
                  for i in range(R)]

    @plsc.parallel_loop(0, _VP // _L, 1, unroll=2,
                        carry=(jnp.int32(0),) * R)
    def csb(j, carries):
        hs = [hists[i][pl.ds(j * _L, _L)] for i in range(R)]
        incs = [plsc.cumsum(hs[i]) for i in range(R)]
        sums = [jnp.sum(hs[i], axis=0) for i in range(R)]
        for i in range(R):
            hists[i][pl.ds(j * _L, _L)] = incs[i] - hs[i] + carries[i]
        return tuple(carries[i] + sums[i] for i in range(R))

    for i in range(R):
        out_idxs[i][pl.ds(_KPAD - _L, _L)] = jnp.zeros((_L,), jnp.int32)
        sel_toks[i][pl.ds(_KPAD - _L, _L)] = jnp.zeros((_L,), jnp.int32)

    def p2(c, _):
        off = c * _L
        svec = lax.iota(jnp.int32, _L) + off
        rs = [rank_rows[i][pl.ds(off, _L)] for i in range(R)]
        toks = [x_rows[i][pl.ds(off, _L)] for i in range(R)]
        bases = [plsc.load_gather(hists[i], [rs[i]]) for i in range(R)]
        scans = [plsc.scan_count(rs[i]) for i in range(R)]
        for i in range(R):
            cnt, lastr = scans[i]
            pos = bases[i] + cnt - 1
            plsc.store_scatter(hists[i], [rs[i]], pos + 1, mask=lastr)
            sel = pos < _K
            plsc.store_scatter(out_idxs[i], [pos], svec, mask=sel)
            plsc.store_scatter(sel_toks[i], [pos], toks[i], mask=sel)
        return 0
    lax.fori_loop(0, _S // _L, p2, 0)

    @plsc.parallel_loop(0, _KPAD // _L, 1, unroll=2)
    def pcnt(c):
        off = c * _L
        lane = lax.iota(jnp.int32, _L)
        valid = (lane + off) < _K
        toks = [sel_toks[i][pl.ds(off, _L)] for i in range(R)]
        tokms = [jnp.where(valid, toks[i], _VP + lane) for i in range(R)]
        scans = [plsc.scan_count(tokms[i]) for i in range(R)]
        for i in range(R):
            cnts, lasts = scans[i]
            wm = jnp.logical_and(lasts, valid)
            plsc.addupdate_scatter(cnt_toks[i], [toks[i]],
                                   cnts.astype(jnp.float32), mask=wm)

    out_dmas = [pltpu.async_copy(out_idxs[i], topidx_hbm.at[b0 + i], sem)
                for i in range(R)]
    out_dmas += [pltpu.async_copy(cnt_toks[i], cnt_hbm.at[b0 + i], sem)
                 for i in range(R)]
    for d in score_dmas + out_dmas:
        d.wait()


def _make_sc_select():
    mesh = plsc.VectorSubcoreMesh(core_axis_name="c", subcore_axis_name="s", num_cores=_NC)
    R = _ROWS_PER
    return pl.kernel(
        _sc_body,
        out_type=(
            jax.ShapeDtypeStruct((_B, _S), jnp.float32),
            jax.ShapeDtypeStruct((_B, _KPAD), jnp.int32),
            jax.ShapeDtypeStruct((_B, _VP), jnp.float32),
        ),
        mesh=mesh,
        compiler_params=pltpu.CompilerParams(needs_layout_passes=False),
        scratch_types=(
            [pltpu.VMEM((_VP,), jnp.int32),
             pltpu.VMEM((_VP,), jnp.float32)]
            + [pltpu.VMEM((_S,), jnp.int32) for _ in range(R)]
            + [pltpu.VMEM((_S,), jnp.int32) for _ in range(R)]
            + [pltpu.VMEM((_S,), jnp.float32) for _ in range(R)]
            + [pltpu.VMEM((_VP,), jnp.int32) for _ in range(R)]
            + [pltpu.VMEM((_VP,), jnp.float32) for _ in range(R)]
            + [pltpu.VMEM((_KPAD,), jnp.int32) for _ in range(R)]
            + [pltpu.VMEM((_KPAD,), jnp.int32) for _ in range(R)]
            + [pltpu.SemaphoreType.DMA]
        ),
    )


def _head_body(cnt_ref, g_ref, c1_ref, bc1_ref, c2_ref, bc2_ref, pred_ref):
    pooled = jnp.dot(cnt_ref[...], g_ref[...], precision=lax.Precision.HIGHEST,
                     preferred_element_type=jnp.float32) * (1.0 / _K)
    h = jax.nn.relu(jnp.dot(pooled, c1_ref[...], precision=lax.Precision.HIGHEST,
                            preferred_element_type=jnp.float32) + bc1_ref[...])
    out = jnp.sum(h * c2_ref[...], axis=1, keepdims=True) + bc2_ref[...]
    pred_ref[...] = jax.nn.sigmoid(out)


_head_kernel = pl.pallas_call(
    _head_body,
    out_shape=jax.ShapeDtypeStruct((_B, 1), jnp.float32),
)


def kernel(x, table, W1, b1, W2, b2, W3, b3, A1, bA1, A2, bA2, C1, bC1, C2,
           bC2):
    x = x.astype(jnp.int32)
    tab_p = jnp.pad(table, ((0, _VP - _VOCAB), (0, 0)))
    h = jax.nn.relu(tab_p @ W1 + b1)
    h = jax.nn.relu(h @ W2 + b2)
    score_col = jax.nn.sigmoid(h @ W3 + b3)
    rank_col, g = _vocab_kernel(score_col, tab_p, A1, bA1.reshape(1, -1),
                                A2, bA2.reshape(1, -1))
    scores, top_idx_pad, cnt = _make_sc_select()(
        x, rank_col.reshape(_VP), score_col.reshape(_VP))
    top_idx = top_idx_pad[:, :_K]
    pred = _head_kernel(cnt, g, C1, bC1.reshape(1, -1), C2.reshape(1, -1),
                        bC2.reshape(1, 1))
    return pred.reshape(_B), top_idx, scores


# speedup vs baseline: 1.0310x; 1.0310x over previous
import functools

import jax
import jax.numpy as jnp
from jax import lax
from jax.experimental import pallas as pl
from jax.experimental.pallas import tpu as pltpu
from jax.experimental.pallas import tpu_sc as plsc

_VOCAB = 1000
_VP = 1024
_D = 512
_B = 128
_S = 2048
_K = 204
_KPAD = 208
_NC = 2
_NW = 16 * _NC
_ROWS_PER = _B // _NW
_L = 16


def _vocab_body(sc_ref, tab_ref, a1_ref, ba1_ref, a2_ref, ba2_ref,
                rank_ref, g_ref):
    sc = sc_ref[...]
    bits = lax.bitcast_convert_type(sc, jnp.int32)
    bits_r = jnp.transpose(bits, (1, 0))
    gt = bits > bits_r
    real_row = lax.broadcasted_iota(jnp.int32, (_VP, _VP), 1) < _VOCAB
    nsmaller = jnp.sum(jnp.where(gt & real_row, 1, 0).astype(jnp.int32),
                       axis=1, keepdims=True)
    rank_ref[...] = (_VOCAB - 1) - nsmaller

    tab = tab_ref[...]
    a1 = jax.nn.relu(jnp.dot(tab, a1_ref[...], precision=lax.Precision.HIGHEST,
                             preferred_element_type=jnp.float32) + ba1_ref[...])
    g_ref[...] = jnp.dot(a1, a2_ref[...], precision=lax.Precision.HIGHEST,
                         preferred_element_type=jnp.float32) + ba2_ref[...]


_vocab_kernel = pl.pallas_call(
    _vocab_body,
    out_shape=(
        jax.ShapeDtypeStruct((_VP, 1), jnp.int32),
        jax.ShapeDtypeStruct((_VP, _D), jnp.float32),
    ),
)


def _sc_body(x_hbm, vrank_hbm, vscore_hbm,
             scores_hbm, topidx_hbm, cnt_hbm,
             *scratch):
    R = _ROWS_PER
    vrank, vscore = scratch[0], scratch[1]
    x_rows = scratch[2:2 + R]
    rank_rows = scratch[2 + R:2 + 2 * R]
    score_rows = scratch[2 + 2 * R:2 + 3 * R]
    hists = scratch[2 + 3 * R:2 + 4 * R]
    cnt_toks = scratch[2 + 4 * R:2 + 5 * R]
    out_idxs = scratch[2 + 5 * R:2 + 6 * R]
    sel_toks = scratch[2 + 6 * R:2 + 7 * R]
    sem = scratch[2 + 7 * R]

    cid = lax.axis_index("c")
    sid = lax.axis_index("s")
    wid = sid * _NC + cid
    b0 = wid * R
    in_dmas = [pltpu.async_copy(x_hbm.at[b0 + i], x_rows[i], sem)
               for i in range(R)]
    pltpu.sync_copy(vrank_hbm, vrank)
    pltpu.sync_copy(vscore_hbm, vscore)
    for d in in_dmas:
        d.wait()

    @plsc.parallel_loop(0, _VP // _L, 1, unroll=4)
    def zero_body(j):
        for i in range(R):
            hists[i][pl.ds(j * _L, _L)] = jnp.zeros((_L,), jnp.int32)
            cnt_toks[i][pl.ds(j * _L, _L)] = jnp.zeros((_L,), jnp.float32)

    @plsc.parallel_loop(0, _S // _L, 1, unroll=2)
    def p1(c):
        off = c * _L
        toks = [x_rows[i][pl.ds(off, _L)] for i in range(R)]
        rs = [plsc.load_gather(vrank, [toks[i]]) for i in range(R)]
        scs = [plsc.load_gather(vscore, [toks[i]]) for i in range(R)]
        scans = [plsc.scan_count(rs[i]) for i in range(R)]
        for i in range(R):
            rank_rows[i][pl.ds(off, _L)] = rs[i]
        for i in range(R):
            score_rows[i][pl.ds(off, _L)] = scs[i]
        for i in range(R):
            cnt, last = scans[i]
            plsc.addupdate_scatter(hists[i], [rs[i]], cnt, mask=last)
    score_dmas = [pltpu.async_copy(score_rows[i], scores_hbm.at[b0 + i], sem)
                  for i in range(R)]

    @plsc.parallel_loop(0, _VP // _L, 1, unroll=2,
                        carry=(jnp.int32(0),) * R)
    def csb(j, carries):
        hs = [hists[i][pl.ds(j * _L, _L)] for i in range(R)]
        incs = [plsc.cumsum(hs[i]) for i in range(R)]
        sums = [jnp.sum(hs[i], axis=0) for i in range(R)]
        for i in range(R):
            hists[i][pl.ds(j * _L, _L)] = incs[i] - hs[i] + carries[i]
        return tuple(carries[i] + sums[i] for i in range(R))

    for i in range(R):
        out_idxs[i][pl.ds(_KPAD - _L, _L)] = jnp.zeros((_L,), jnp.int32)
        sel_toks[i][pl.ds(_KPAD - _L, _L)] = jnp.zeros((_L,), jnp.int32)

    def p2(c, _):
        off = c * _L
        svec = lax.iota(jnp.int32, _L) + off
        rs = [rank_rows[i][pl.ds(off, _L)] for i in range(R)]
        toks = [x_rows[i][pl.ds(off, _L)] for i in range(R)]
        bases = [plsc.load_gather(hists[i], [rs[i]]) for i in range(R)]
        scans = [plsc.scan_count(rs[i]) for i in range(R)]
        for i in range(R):
            cnt, lastr = scans[i]
            pos = bases[i] + cnt - 1
            plsc.store_scatter(hists[i], [rs[i]], pos + 1, mask=lastr)
            sel = pos < _K
            plsc.store_scatter(out_idxs[i], [pos], svec, mask=sel)
            plsc.store_scatter(sel_toks[i], [pos], toks[i], mask=sel)
        return 0
    lax.fori_loop(0, _S // _L, p2, 0)

    @plsc.parallel_loop(0, _KPAD // _L, 1, unroll=2)
    def pcnt(c):
        off = c * _L
        lane = lax.iota(jnp.int32, _L)
        valid = (lane + off) < _K
        toks = [sel_toks[i][pl.ds(off, _L)] for i in range(R)]
        tokms = [jnp.where(valid, toks[i], _VP + lane) for i in range(R)]
        scans = [plsc.scan_count(tokms[i]) for i in range(R)]
        for i in range(R):
            cnts, lasts = scans[i]
            wm = jnp.logical_and(lasts, valid)
            plsc.addupdate_scatter(cnt_toks[i], [toks[i]],
                                   cnts.astype(jnp.float32), mask=wm)

    out_dmas = [pltpu.async_copy(out_idxs[i], topidx_hbm.at[b0 + i], sem)
                for i in range(R)]
    out_dmas += [pltpu.async_copy(cnt_toks[i], cnt_hbm.at[b0 + i], sem)
                 for i in range(R)]
    for d in score_dmas + out_dmas:
        d.wait()


def _make_sc_select():
    mesh = plsc.VectorSubcoreMesh(core_axis_name="c", subcore_axis_name="s", num_cores=_NC)
    R = _ROWS_PER
    return pl.kernel(
        _sc_body,
        out_type=(
            jax.ShapeDtypeStruct((_B, _S), jnp.float32),
            jax.ShapeDtypeStruct((_B, _KPAD), jnp.int32),
            jax.ShapeDtypeStruct((_B, _VP), jnp.float32),
        ),
        mesh=mesh,
        compiler_params=pltpu.CompilerParams(needs_layout_passes=False),
        scratch_types=(
            [pltpu.VMEM((_VP,), jnp.int32),
             pltpu.VMEM((_VP,), jnp.float32)]
            + [pltpu.VMEM((_S,), jnp.int32) for _ in range(R)]
            + [pltpu.VMEM((_S,), jnp.int32) for _ in range(R)]
            + [pltpu.VMEM((_S,), jnp.float32) for _ in range(R)]
            + [pltpu.VMEM((_VP,), jnp.int32) for _ in range(R)]
            + [pltpu.VMEM((_VP,), jnp.float32) for _ in range(R)]
            + [pltpu.VMEM((_KPAD,), jnp.int32) for _ in range(R)]
            + [pltpu.VMEM((_KPAD,), jnp.int32) for _ in range(R)]
            + [pltpu.SemaphoreType.DMA]
        ),
    )


def _head_body(cnt_ref, g_ref, c1_ref, bc1_ref, c2_ref, bc2_ref, pred_ref):
    pooled = jnp.dot(cnt_ref[...], g_ref[...], precision=lax.Precision.HIGHEST,
                     preferred_element_type=jnp.float32) * (1.0 / _K)
    h = jax.nn.relu(jnp.dot(pooled, c1_ref[...], precision=lax.Precision.HIGHEST,
                            preferred_element_type=jnp.float32) + bc1_ref[...])
    out = jnp.sum(h * c2_ref[...], axis=1, keepdims=True) + bc2_ref[...]
    pred_ref[...] = jax.nn.sigmoid(out)


_head_kernel = pl.pallas_call(
    _head_body,
    out_shape=jax.ShapeDtypeStruct((_B, 1), jnp.float32),
)


def kernel(x, table, W1, b1, W2, b2, W3, b3, A1, bA1, A2, bA2, C1, bC1, C2,
           bC2):
    x = x.astype(jnp.int32)
    tab_p = jnp.pad(table, ((0, _VP - _VOCAB), (0, 0)))
    h = jax.nn.relu(tab_p @ W1 + b1)
    h = jax.nn.relu(h @ W2 + b2)
    score_col = jax.nn.sigmoid(h @ W3 + b3)
    rank_col, g = _vocab_kernel(score_col, tab_p, A1, bA1.reshape(1, -1),
                                A2, bA2.reshape(1, -1))
    scores, top_idx_pad, cnt = _make_sc_select()(
        x, rank_col.reshape(_VP), score_col.reshape(_VP))
    top_idx = top_idx_pad[:, :_K]
    pred = _head_kernel(cnt, g, C1, bC1.reshape(1, -1), C2.reshape(1, -1),
                        bC2.reshape(1, 1))
    return pred.reshape(_B), top_idx, scores


# speedup vs baseline: 1.0737x; 1.0414x over previous
import functools

import jax
import jax.numpy as jnp
from jax import lax
from jax.experimental import pallas as pl
from jax.experimental.pallas import tpu as pltpu
from jax.experimental.pallas import tpu_sc as plsc

_VOCAB = 1000
_VP = 1024
_D = 512
_B = 128
_S = 2048
_K = 204
_KPAD = 208
_NC = 2
_NW = 16 * _NC
_ROWS_PER = _B // _NW
_L = 16


def _vocab_body(sc_ref, rank_ref):
    sc = sc_ref[...]
    bits = lax.bitcast_convert_type(sc, jnp.int32)
    bits_r = jnp.transpose(bits, (1, 0))
    gt = bits > bits_r
    real_row = lax.broadcasted_iota(jnp.int32, (_VP, _VP), 1) < _VOCAB
    nsmaller = jnp.sum(jnp.where(gt & real_row, 1, 0).astype(jnp.int32),
                       axis=1, keepdims=True)
    rank_ref[...] = (_VOCAB - 1) - nsmaller


_vocab_kernel = pl.pallas_call(
    _vocab_body,
    out_shape=jax.ShapeDtypeStruct((_VP, 1), jnp.int32),
)


def _sc_body(x_hbm, vrank_hbm, vscore_hbm,
             scores_hbm, topidx_hbm, cnt_hbm,
             *scratch):
    R = _ROWS_PER
    vrank, vscore = scratch[0], scratch[1]
    x_rows = scratch[2:2 + R]
    rank_rows = scratch[2 + R:2 + 2 * R]
    score_rows = scratch[2 + 2 * R:2 + 3 * R]
    hists = scratch[2 + 3 * R:2 + 4 * R]
    cnt_toks = scratch[2 + 4 * R:2 + 5 * R]
    out_idxs = scratch[2 + 5 * R:2 + 6 * R]
    sel_toks = scratch[2 + 6 * R:2 + 7 * R]
    sem = scratch[2 + 7 * R]

    cid = lax.axis_index("c")
    sid = lax.axis_index("s")
    wid = sid * _NC + cid
    b0 = wid * R
    in_dmas = [pltpu.async_copy(x_hbm.at[b0 + i], x_rows[i], sem)
               for i in range(R)]
    pltpu.sync_copy(vrank_hbm, vrank)
    pltpu.sync_copy(vscore_hbm, vscore)
    for d in in_dmas:
        d.wait()

    @plsc.parallel_loop(0, _VP // _L, 1, unroll=4)
    def zero_body(j):
        for i in range(R):
            hists[i][pl.ds(j * _L, _L)] = jnp.zeros((_L,), jnp.int32)
            cnt_toks[i][pl.ds(j * _L, _L)] = jnp.zeros((_L,), jnp.float32)

    @plsc.parallel_loop(0, _S // _L, 1, unroll=2)
    def p1(c):
        off = c * _L
        toks = [x_rows[i][pl.ds(off, _L)] for i in range(R)]
        rs = [plsc.load_gather(vrank, [toks[i]]) for i in range(R)]
        scs = [plsc.load_gather(vscore, [toks[i]]) for i in range(R)]
        scans = [plsc.scan_count(rs[i]) for i in range(R)]
        for i in range(R):
            rank_rows[i][pl.ds(off, _L)] = rs[i]
        for i in range(R):
            score_rows[i][pl.ds(off, _L)] = scs[i]
        for i in range(R):
            cnt, last = scans[i]
            plsc.addupdate_scatter(hists[i], [rs[i]], cnt, mask=last)
    score_dmas = [pltpu.async_copy(score_rows[i], scores_hbm.at[b0 + i], sem)
                  for i in range(R)]

    @plsc.parallel_loop(0, _VP // _L, 1, unroll=2,
                        carry=(jnp.int32(0),) * R)
    def csb(j, carries):
        hs = [hists[i][pl.ds(j * _L, _L)] for i in range(R)]
        incs = [plsc.cumsum(hs[i]) for i in range(R)]
        sums = [jnp.sum(hs[i], axis=0) for i in range(R)]
        for i in range(R):
            hists[i][pl.ds(j * _L, _L)] = incs[i] - hs[i] + carries[i]
        return tuple(carries[i] + sums[i] for i in range(R))

    for i in range(R):
        out_idxs[i][pl.ds(_KPAD - _L, _L)] = jnp.zeros((_L,), jnp.int32)
        sel_toks[i][pl.ds(_KPAD - _L, _L)] = jnp.zeros((_L,), jnp.int32)

    def p2(c, _):
        off = c * _L
        svec = lax.iota(jnp.int32, _L) + off
        rs = [rank_rows[i][pl.ds(off, _L)] for i in range(R)]
        toks = [x_rows[i][pl.ds(off, _L)] for i in range(R)]
        bases = [plsc.load_gather(hists[i], [rs[i]]) for i in range(R)]
        scans = [plsc.scan_count(rs[i]) for i in range(R)]
        for i in range(R):
            cnt, lastr = scans[i]
            pos = bases[i] + cnt - 1
            plsc.store_scatter(hists[i], [rs[i]], pos + 1, mask=lastr)
            sel = pos < _K
            plsc.store_scatter(out_idxs[i], [pos], svec, mask=sel)
            plsc.store_scatter(sel_toks[i], [pos], toks[i], mask=sel)
        return 0
    lax.fori_loop(0, _S // _L, p2, 0)

    @plsc.parallel_loop(0, _KPAD // _L, 1, unroll=2)
    def pcnt(c):
        off = c * _L
        lane = lax.iota(jnp.int32, _L)
        valid = (lane + off) < _K
        toks = [sel_toks[i][pl.ds(off, _L)] for i in range(R)]
        tokms = [jnp.where(valid, toks[i], _VP + lane) for i in range(R)]
        scans = [plsc.scan_count(tokms[i]) for i in range(R)]
        for i in range(R):
            cnts, lasts = scans[i]
            wm = jnp.logical_and(lasts, valid)
            plsc.addupdate_scatter(cnt_toks[i], [toks[i]],
                                   cnts.astype(jnp.float32), mask=wm)

    out_dmas = [pltpu.async_copy(out_idxs[i], topidx_hbm.at[b0 + i], sem)
                for i in range(R)]
    out_dmas += [pltpu.async_copy(cnt_toks[i], cnt_hbm.at[b0 + i], sem)
                 for i in range(R)]
    for d in score_dmas + out_dmas:
        d.wait()


def _make_sc_select():
    mesh = plsc.VectorSubcoreMesh(core_axis_name="c", subcore_axis_name="s", num_cores=_NC)
    R = _ROWS_PER
    return pl.kernel(
        _sc_body,
        out_type=(
            jax.ShapeDtypeStruct((_B, _S), jnp.float32),
            jax.ShapeDtypeStruct((_B, _KPAD), jnp.int32),
            jax.ShapeDtypeStruct((_B, _VP), jnp.float32),
        ),
        mesh=mesh,
        compiler_params=pltpu.CompilerParams(needs_layout_passes=False),
        scratch_types=(
            [pltpu.VMEM((_VP,), jnp.int32),
             pltpu.VMEM((_VP,), jnp.float32)]
            + [pltpu.VMEM((_S,), jnp.int32) for _ in range(R)]
            + [pltpu.VMEM((_S,), jnp.int32) for _ in range(R)]
            + [pltpu.VMEM((_S,), jnp.float32) for _ in range(R)]
            + [pltpu.VMEM((_VP,), jnp.int32) for _ in range(R)]
            + [pltpu.VMEM((_VP,), jnp.float32) for _ in range(R)]
            + [pltpu.VMEM((_KPAD,), jnp.int32) for _ in range(R)]
            + [pltpu.VMEM((_KPAD,), jnp.int32) for _ in range(R)]
            + [pltpu.SemaphoreType.DMA]
        ),
    )


def _head_body(cnt_ref, tab_ref, a1_ref, ba1_ref, a2_ref, ba2_ref,
               c1_ref, bc1_ref, c2_ref, bc2_ref, pred_ref):
    a1 = jax.nn.relu(jnp.dot(tab_ref[...], a1_ref[...],
                             precision=lax.Precision.HIGHEST,
                             preferred_element_type=jnp.float32) + ba1_ref[...])
    g = jnp.dot(a1, a2_ref[...], precision=lax.Precision.HIGHEST,
                preferred_element_type=jnp.float32) + ba2_ref[...]
    pooled = jnp.dot(cnt_ref[...], g, precision=lax.Precision.HIGHEST,
                     preferred_element_type=jnp.float32) * (1.0 / _K)
    h = jax.nn.relu(jnp.dot(pooled, c1_ref[...], precision=lax.Precision.HIGHEST,
                            preferred_element_type=jnp.float32) + bc1_ref[...])
    out = jnp.sum(h * c2_ref[...], axis=1, keepdims=True) + bc2_ref[...]
    pred_ref[...] = jax.nn.sigmoid(out)


_head_kernel = pl.pallas_call(
    _head_body,
    out_shape=jax.ShapeDtypeStruct((_B, 1), jnp.float32),
)


def kernel(x, table, W1, b1, W2, b2, W3, b3, A1, bA1, A2, bA2, C1, bC1, C2,
           bC2):
    x = x.astype(jnp.int32)
    tab_p = jnp.pad(table, ((0, _VP - _VOCAB), (0, 0)))
    h = jax.nn.relu(tab_p @ W1 + b1)
    h = jax.nn.relu(h @ W2 + b2)
    score_col = jax.nn.sigmoid(h @ W3 + b3)
    rank_col = _vocab_kernel(score_col)
    scores, top_idx_pad, cnt = _make_sc_select()(
        x, rank_col.reshape(_VP), score_col.reshape(_VP))
    top_idx = top_idx_pad[:, :_K]
    pred = _head_kernel(cnt, tab_p, A1, bA1.reshape(1, -1), A2,
                        bA2.reshape(1, -1), C1, bC1.reshape(1, -1),
                        C2.reshape(1, -1), bC2.reshape(1, 1))
    return pred.reshape(_B), top_idx, scores


# speedup vs baseline: 1.0757x; 1.0019x over previous
import functools

import jax
import jax.numpy as jnp
from jax import lax
from jax.experimental import pallas as pl
from jax.experimental.pallas import tpu as pltpu
from jax.experimental.pallas import tpu_sc as plsc

_VOCAB = 1000
_VP = 1024
_D = 512
_B = 128
_S = 2048
_K = 204
_KPAD = 208
_NC = 2
_NW = 16 * _NC
_ROWS_PER = _B // _NW
_L = 16


def _vocab_body(sc_ref, rank_ref):
    sc = sc_ref[...]
    bits = lax.bitcast_convert_type(sc, jnp.int32)
    bits_r = jnp.transpose(bits, (1, 0))
    gt = bits > bits_r
    real_row = lax.broadcasted_iota(jnp.int32, (_VP, _VP), 1) < _VOCAB
    nsmaller = jnp.sum(jnp.where(gt & real_row, 1, 0).astype(jnp.int32),
                       axis=1, keepdims=True)
    rank_ref[...] = (_VOCAB - 1) - nsmaller


_vocab_kernel = pl.pallas_call(
    _vocab_body,
    out_shape=jax.ShapeDtypeStruct((_VP, 1), jnp.int32),
)


def _sc_body(x_hbm, vrank_hbm, vscore_hbm,
             scores_hbm, topidx_hbm, cnt_hbm,
             *scratch):
    R = _ROWS_PER
    vrank, vscore = scratch[0], scratch[1]
    x_rows = scratch[2:2 + R]
    rank_rows = scratch[2 + R:2 + 2 * R]
    score_rows = scratch[2 + 2 * R:2 + 3 * R]
    hists = scratch[2 + 3 * R:2 + 4 * R]
    cnt_toks = scratch[2 + 4 * R:2 + 5 * R]
    out_idxs = scratch[2 + 5 * R:2 + 6 * R]
    sel_toks = scratch[2 + 6 * R:2 + 7 * R]
    sem = scratch[2 + 7 * R]

    cid = lax.axis_index("c")
    sid = lax.axis_index("s")
    wid = sid * _NC + cid
    b0 = wid * R
    in_dmas = [pltpu.async_copy(x_hbm.at[b0 + i], x_rows[i], sem)
               for i in range(R)]
    pltpu.sync_copy(vrank_hbm, vrank)
    pltpu.sync_copy(vscore_hbm, vscore)
    for d in in_dmas:
        d.wait()

    @plsc.parallel_loop(0, _VP // _L, 1, unroll=4)
    def zero_body(j):
        for i in range(R):
            hists[i][pl.ds(j * _L, _L)] = jnp.zeros((_L,), jnp.int32)
            cnt_toks[i][pl.ds(j * _L, _L)] = jnp.zeros((_L,), jnp.float32)

    @plsc.parallel_loop(0, _S // _L, 1, unroll=2)
    def p1(c):
        off = c * _L
        toks = [x_rows[i][pl.ds(off, _L)] for i in range(R)]
        rs = [plsc.load_gather(vrank, [toks[i]]) for i in range(R)]
        scs = [plsc.load_gather(vscore, [toks[i]]) for i in range(R)]
        scans = [plsc.scan_count(rs[i]) for i in range(R)]
        for i in range(R):
            rank_rows[i][pl.ds(off, _L)] = rs[i]
        for i in range(R):
            score_rows[i][pl.ds(off, _L)] = scs[i]
        for i in range(R):
            cnt, last = scans[i]
            plsc.addupdate_scatter(hists[i], [rs[i]], cnt, mask=last)
    score_dmas = [pltpu.async_copy(score_rows[i], scores_hbm.at[b0 + i], sem)
                  for i in range(R)]

    @plsc.parallel_loop(0, _VP // _L, 1, unroll=2,
                        carry=(jnp.int32(0),) * R)
    def csb(j, carries):
        hs = [hists[i][pl.ds(j * _L, _L)] for i in range(R)]
        incs = [plsc.cumsum(hs[i]) for i in range(R)]
        sums = [jnp.sum(hs[i], axis=0) for i in range(R)]
        for i in range(R):
            hists[i][pl.ds(j * _L, _L)] = incs[i] - hs[i] + carries[i]
        return tuple(carries[i] + sums[i] for i in range(R))

    for i in range(R):
        out_idxs[i][pl.ds(_KPAD - _L, _L)] = jnp.zeros((_L,), jnp.int32)
        sel_toks[i][pl.ds(_KPAD - _L, _L)] = jnp.zeros((_L,), jnp.int32)

    def p2(c, _):
        off = c * _L
        svec = lax.iota(jnp.int32, _L) + off
        rs = [rank_rows[i][pl.ds(off, _L)] for i in range(R)]
        toks = [x_rows[i][pl.ds(off, _L)] for i in range(R)]
        bases = [plsc.load_gather(hists[i], [rs[i]]) for i in range(R)]
        scans = [plsc.scan_count(rs[i]) for i in range(R)]
        for i in range(R):
            cnt, lastr = scans[i]
            pos = bases[i] + cnt - 1
            plsc.store_scatter(hists[i], [rs[i]], pos + 1, mask=lastr)
            sel = pos < _K
            plsc.store_scatter(out_idxs[i], [pos], svec, mask=sel)
            plsc.store_scatter(sel_toks[i], [pos], toks[i], mask=sel)
        return 0
    lax.fori_loop(0, _S // _L, p2, 0)

    @plsc.parallel_loop(0, _KPAD // _L, 1, unroll=2)
    def pcnt(c):
        off = c * _L
        lane = lax.iota(jnp.int32, _L)
        valid = (lane + off) < _K
        toks = [sel_toks[i][pl.ds(off, _L)] for i in range(R)]
        tokms = [jnp.where(valid, toks[i], _VP + lane) for i in range(R)]
        scans = [plsc.scan_count(tokms[i]) for i in range(R)]
        for i in range(R):
            cnts, lasts = scans[i]
            wm = jnp.logical_and(lasts, valid)
            plsc.addupdate_scatter(cnt_toks[i], [toks[i]],
                                   cnts.astype(jnp.float32), mask=wm)

    out_dmas = [pltpu.async_copy(out_idxs[i], topidx_hbm.at[b0 + i], sem)
                for i in range(R)]
    out_dmas += [pltpu.async_copy(cnt_toks[i], cnt_hbm.at[b0 + i], sem)
                 for i in range(R)]
    for d in score_dmas + out_dmas:
        d.wait()


def _make_sc_select():
    mesh = plsc.VectorSubcoreMesh(core_axis_name="c", subcore_axis_name="s", num_cores=_NC)
    R = _ROWS_PER
    return pl.kernel(
        _sc_body,
        out_type=(
            jax.ShapeDtypeStruct((_B, _S), jnp.float32),
            jax.ShapeDtypeStruct((_B, _KPAD), jnp.int32),
            jax.ShapeDtypeStruct((_B, _VP), jnp.float32),
        ),
        mesh=mesh,
        compiler_params=pltpu.CompilerParams(needs_layout_passes=False),
        scratch_types=(
            [pltpu.VMEM((_VP,), jnp.int32),
             pltpu.VMEM((_VP,), jnp.float32)]
            + [pltpu.VMEM((_S,), jnp.int32) for _ in range(R)]
            + [pltpu.VMEM((_S,), jnp.int32) for _ in range(R)]
            + [pltpu.VMEM((_S,), jnp.float32) for _ in range(R)]
            + [pltpu.VMEM((_VP,), jnp.int32) for _ in range(R)]
            + [pltpu.VMEM((_VP,), jnp.float32) for _ in range(R)]
            + [pltpu.VMEM((_KPAD,), jnp.int32) for _ in range(R)]
            + [pltpu.VMEM((_KPAD,), jnp.int32) for _ in range(R)]
            + [pltpu.SemaphoreType.DMA]
        ),
    )


def _head_body(cnt_ref, tab_ref, a1_ref, ba1_ref, a2_ref, ba2_ref,
               c1_ref, bc1_ref, c2_ref, bc2_ref, pred_ref):
    a1 = jax.nn.relu(jnp.dot(tab_ref[...], a1_ref[...],
                             precision=lax.Precision.HIGHEST,
                             preferred_element_type=jnp.float32) + ba1_ref[...])
    g = jnp.dot(a1, a2_ref[...], precision=lax.Precision.HIGHEST,
                preferred_element_type=jnp.float32) + ba2_ref[...]
    pooled = jnp.dot(cnt_ref[...][:, :_VOCAB], g,
                     precision=lax.Precision.HIGHEST,
                     preferred_element_type=jnp.float32) * (1.0 / _K)
    h = jax.nn.relu(jnp.dot(pooled, c1_ref[...], precision=lax.Precision.HIGHEST,
                            preferred_element_type=jnp.float32) + bc1_ref[...])
    out = jnp.sum(h * c2_ref[...], axis=1, keepdims=True) + bc2_ref[...]
    pred_ref[...] = jax.nn.sigmoid(out)


_head_kernel = pl.pallas_call(
    _head_body,
    out_shape=jax.ShapeDtypeStruct((_B, 1), jnp.float32),
)


def kernel(x, table, W1, b1, W2, b2, W3, b3, A1, bA1, A2, bA2, C1, bC1, C2,
           bC2):
    x = x.astype(jnp.int32)
    h = jax.nn.relu(table @ W1 + b1)
    h = jax.nn.relu(h @ W2 + b2)
    score_col = jnp.pad(jax.nn.sigmoid(h @ W3 + b3),
                        ((0, _VP - _VOCAB), (0, 0)),
                        constant_values=1.0)
    rank_col = _vocab_kernel(score_col)
    scores, top_idx_pad, cnt = _make_sc_select()(
        x, rank_col.reshape(_VP), score_col.reshape(_VP))
    top_idx = top_idx_pad[:, :_K]
    pred = _head_kernel(cnt, table, A1, bA1.reshape(1, -1), A2,
                        bA2.reshape(1, -1), C1, bC1.reshape(1, -1),
                        C2.reshape(1, -1), bC2.reshape(1, 1))
    return pred.reshape(_B), top_idx, scores
